# baseline (speedup 1.0000x reference)
import functools

import jax
import jax.numpy as jnp
from jax import lax
from jax.experimental import pallas as pl
from jax.experimental.pallas import tpu as pltpu
from jax.experimental.pallas import tpu_sc as plsc

_BETA = 0.25
_CHUNK = 512
_GPAD = 128


def _dist_argmin_body(z_ref, emb_ref, zn_ref, en_ref, idx_ref):
    i = pl.program_id(0)
    n_embed = emb_ref.shape[0]
    dim = emb_ref.shape[1]
    m = z_ref.shape[0]

    @pl.when(i == 0)
    def _():
        e = emb_ref[...]
        en = jnp.sqrt(jnp.sum(e * e, axis=1, keepdims=True))
        en_ref[:, 0:dim] = e / jnp.maximum(en, 1e-12)

    z = z_ref[...]
    zn_norm = jnp.sqrt(jnp.sum(z * z, axis=1, keepdims=True))
    zn = z / jnp.maximum(zn_norm, 1e-12)
    zn_ref[...] = zn

    zsq = jnp.sum(zn * zn, axis=1, keepdims=True)
    zsq_t = jnp.transpose(zsq)

    n_chunks = n_embed // _CHUNK

    def body(c, carry):
        best_val, best_idx = carry
        en_c = en_ref[pl.ds(c * _CHUNK, _CHUNK), 0:dim]
        esq = jnp.sum(en_c * en_c, axis=1, keepdims=True)
        mm = lax.dot_general(
            en_c, zn, (((1,), (1,)), ((), ())),
            preferred_element_type=jnp.float32)
        d = (esq + zsq_t) - 2.0 * mm
        cmin = jnp.min(d, axis=0, keepdims=True)
        row_ids = lax.broadcasted_iota(jnp.int32, (_CHUNK, m), 0) + c * _CHUNK
        cidx = jnp.min(jnp.where(d == cmin, row_ids, n_embed),
                       axis=0, keepdims=True)
        new_idx = jnp.where(cmin < best_val, cidx, best_idx)
        new_val = jnp.minimum(best_val, cmin)
        return new_val, new_idx

    init = (jnp.full((1, m), jnp.inf, jnp.float32),
            jnp.zeros((1, m), jnp.int32))
    _, best_idx = lax.fori_loop(0, n_chunks, body, init)
    idx_ref[0] = best_idx


def _dist_argmin(z_flat, embedding):
    b, dim = z_flat.shape
    n_embed = embedding.shape[0]
    nb = 8
    m = b // nb
    return pl.pallas_call(
        _dist_argmin_body,
        grid=(nb,),
        in_specs=[
            pl.BlockSpec((m, dim), lambda i: (i, 0)),
            pl.BlockSpec((n_embed, dim), lambda i: (0, 0)),
        ],
        out_specs=[
            pl.BlockSpec((m, dim), lambda i: (i, 0)),
            pl.BlockSpec((n_embed, _GPAD), lambda i: (0, 0)),
            pl.BlockSpec((1, 1, m), lambda i: (i, 0, 0)),
        ],
        out_shape=[
            jax.ShapeDtypeStruct((b, dim), jnp.float32),
            jax.ShapeDtypeStruct((n_embed, _GPAD), jnp.float32),
            jax.ShapeDtypeStruct((nb, 1, m), jnp.int32),
        ],
    )(z_flat, embedding)


def _sc_gather(en, idx_flat):
    b = idx_flat.shape[0]
    dim = en.shape[1]
    info = plsc.get_sparse_core_info()
    nc, ns = info.num_cores, info.num_subcores
    nw = nc * ns
    b_per_w = b // nw

    mesh = plsc.VectorSubcoreMesh(core_axis_name="c", subcore_axis_name="s")

    @functools.partial(
        pl.kernel,
        mesh=mesh,
        out_type=jax.ShapeDtypeStruct((b, dim), jnp.float32),
        scratch_types=[
            pltpu.VMEM((b_per_w,), jnp.int32),
            pltpu.VMEM((b_per_w, dim), jnp.float32),
            pltpu.SemaphoreType.DMA,
        ],
    )
    def k(en_hbm, idx_hbm, out_hbm, idx_v, rows_v, sem):
        wid = lax.axis_index("s") * nc + lax.axis_index("c")
        base = wid * b_per_w
        pltpu.sync_copy(idx_hbm.at[pl.ds(base, b_per_w)], idx_v)
        pltpu.async_copy(en_hbm.at[idx_v], rows_v, sem).wait()
        pltpu.sync_copy(rows_v, out_hbm.at[pl.ds(base, b_per_w)])

    return k(en, idx_flat)


def _out_loss_body(z_ref, zn_ref, zq_ref, out_ref, loss_ref):
    z = z_ref[...]
    dim = z_ref.shape[1]
    zq = zq_ref[:, 0:dim]
    out_ref[...] = z + (zq - z)
    diff = zq - zn_ref[...]
    mean = jnp.sum(diff * diff) / jnp.float32(diff.size)
    loss_ref[0, 0] = _BETA * mean + mean


def _out_loss(z_flat, zn, zq_pad):
    b, dim = z_flat.shape
    return pl.pallas_call(
        _out_loss_body,
        out_specs=[
            pl.BlockSpec((b, dim), lambda: (0, 0)),
            pl.BlockSpec(memory_space=pltpu.SMEM),
        ],
        out_shape=[
            jax.ShapeDtypeStruct((b, dim), jnp.float32),
            jax.ShapeDtypeStruct((1, 1), jnp.float32),
        ],
    )(z_flat, zn, zq_pad)


def kernel(z, embedding):
    zshape = z.shape
    dim = zshape[-1]
    z_flat = z.reshape(-1, dim)
    zn, en_pad, idx3 = _dist_argmin(z_flat, embedding)
    idx_flat = idx3.reshape(-1)
    zq_pad = _sc_gather(en_pad, idx_flat)
    z_q_out, loss = _out_loss(z_flat, zn, zq_pad)
    return (z_q_out.reshape(zshape), loss.reshape(()),
            idx3.reshape(zshape[:-1]))


# speedup vs baseline: 1.1478x; 1.1478x over previous
import functools

import jax
import jax.numpy as jnp
from jax import lax
from jax.experimental import pallas as pl
from jax.experimental.pallas import tpu as pltpu
from jax.experimental.pallas import tpu_sc as plsc

_BETA = 0.25
_CHUNK = 2048
_GPAD = 128
_MBLK = 512


def _dist_argmin_body(z_ref, emb_ref, zn_ref, en_ref, idx_ref,
                      esqb_ref, ids_ref):
    i = pl.program_id(0)
    n_embed = emb_ref.shape[0]
    dim = emb_ref.shape[1]
    m = z_ref.shape[0]

    @pl.when(i == 0)
    def _():
        e = emb_ref[...]
        nrm = jnp.sqrt(jnp.sum(e * e, axis=1, keepdims=True))
        en = e / jnp.maximum(nrm, 1e-12)
        en_ref[:, 0:dim] = en
        esq = jnp.sum(en * en, axis=1, keepdims=True)
        esqb_ref[...] = jnp.broadcast_to(esq, (n_embed, m))
        ids_ref[...] = lax.broadcasted_iota(jnp.int32, (_CHUNK, m), 0)

    z = z_ref[...]
    zn_norm = jnp.sqrt(jnp.sum(z * z, axis=1, keepdims=True))
    zn = z / jnp.maximum(zn_norm, 1e-12)
    zn_ref[...] = zn

    zsq = jnp.sum(zn * zn, axis=1, keepdims=True)
    zsq_t = jnp.transpose(zsq)
    zm2 = -2.0 * zn

    n_chunks = n_embed // _CHUNK
    ids0 = ids_ref[...]

    def body(c, carry):
        best_val, best_idx = carry
        en_c = en_ref[pl.ds(c * _CHUNK, _CHUNK), 0:dim]
        mm2 = lax.dot_general(
            en_c, zm2, (((1,), (1,)), ((), ())),
            preferred_element_type=jnp.float32)
        d = (esqb_ref[pl.ds(c * _CHUNK, _CHUNK), :] + zsq_t) + mm2
        cmin = jnp.min(d, axis=0, keepdims=True)
        cloc = jnp.min(jnp.where(d == cmin, ids0, _CHUNK),
                       axis=0, keepdims=True)
        cidx = cloc + c * _CHUNK
        new_idx = jnp.where(cmin < best_val, cidx, best_idx)
        new_val = jnp.minimum(best_val, cmin)
        return new_val, new_idx

    init = (jnp.full((1, m), jnp.inf, jnp.float32),
            jnp.zeros((1, m), jnp.int32))
    _, best_idx = lax.fori_loop(0, n_chunks, body, init)
    idx_ref[0] = best_idx


def _dist_argmin(z_flat, embedding):
    b, dim = z_flat.shape
    n_embed = embedding.shape[0]
    m = _MBLK
    nb = b // m
    return pl.pallas_call(
        _dist_argmin_body,
        grid=(nb,),
        in_specs=[
            pl.BlockSpec((m, dim), lambda i: (i, 0)),
            pl.BlockSpec((n_embed, dim), lambda i: (0, 0)),
        ],
        out_specs=[
            pl.BlockSpec((m, dim), lambda i: (i, 0)),
            pl.BlockSpec((n_embed, _GPAD), lambda i: (0, 0)),
            pl.BlockSpec((1, 1, m), lambda i: (i, 0, 0)),
        ],
        out_shape=[
            jax.ShapeDtypeStruct((b, dim), jnp.float32),
            jax.ShapeDtypeStruct((n_embed, _GPAD), jnp.float32),
            jax.ShapeDtypeStruct((nb, 1, m), jnp.int32),
        ],
        scratch_shapes=[
            pltpu.VMEM((n_embed, m), jnp.float32),
            pltpu.VMEM((_CHUNK, m), jnp.int32),
        ],
    )(z_flat, embedding)


def _sc_gather(en, idx_flat):
    b = idx_flat.shape[0]
    dim = en.shape[1]
    info = plsc.get_sparse_core_info()
    nc, ns = info.num_cores, info.num_subcores
    nw = nc * ns
    b_per_w = b // nw

    mesh = plsc.VectorSubcoreMesh(core_axis_name="c", subcore_axis_name="s")

    @functools.partial(
        pl.kernel,
        mesh=mesh,
        out_type=jax.ShapeDtypeStruct((b, dim), jnp.float32),
        scratch_types=[
            pltpu.VMEM((b_per_w,), jnp.int32),
            pltpu.VMEM((b_per_w, dim), jnp.float32),
            pltpu.SemaphoreType.DMA,
        ],
    )
    def k(en_hbm, idx_hbm, out_hbm, idx_v, rows_v, sem):
        wid = lax.axis_index("s") * nc + lax.axis_index("c")
        base = wid * b_per_w
        pltpu.sync_copy(idx_hbm.at[pl.ds(base, b_per_w)], idx_v)
        pltpu.async_copy(en_hbm.at[idx_v], rows_v, sem).wait()
        pltpu.sync_copy(rows_v, out_hbm.at[pl.ds(base, b_per_w)])

    return k(en, idx_flat)


def _out_loss_body(z_ref, zn_ref, zq_ref, out_ref, loss_ref):
    z = z_ref[...]
    dim = z_ref.shape[1]
    zq = zq_ref[:, 0:dim]
    out_ref[...] = z + (zq - z)
    diff = zq - zn_ref[...]
    mean = jnp.sum(diff * diff) / jnp.float32(diff.size)
    loss_ref[0, 0] = _BETA * mean + mean


def _out_loss(z_flat, zn, zq_pad):
    b, dim = z_flat.shape
    return pl.pallas_call(
        _out_loss_body,
        out_specs=[
            pl.BlockSpec((b, dim), lambda: (0, 0)),
            pl.BlockSpec(memory_space=pltpu.SMEM),
        ],
        out_shape=[
            jax.ShapeDtypeStruct((b, dim), jnp.float32),
            jax.ShapeDtypeStruct((1, 1), jnp.float32),
        ],
    )(z_flat, zn, zq_pad)


def kernel(z, embedding):
    zshape = z.shape
    dim = zshape[-1]
    z_flat = z.reshape(-1, dim)
    zn, en_pad, idx3 = _dist_argmin(z_flat, embedding)
    idx_flat = idx3.reshape(-1)
    zq_pad = _sc_gather(en_pad, idx_flat)
    z_q_out, loss = _out_loss(z_flat, zn, zq_pad)
    return (z_q_out.reshape(zshape), loss.reshape(()),
            idx3.reshape(zshape[:-1]))


# speedup vs baseline: 1.8977x; 1.6533x over previous
import functools

import jax
import jax.numpy as jnp
from jax import lax
from jax.experimental import pallas as pl
from jax.experimental.pallas import tpu as pltpu
from jax.experimental.pallas import tpu_sc as plsc

_BETA = 0.25
_CHUNK = 2048
_GPAD = 128


def _dist_argmin_body(zt_ref, et_ref, znt_ref, en_ref, idx2_ref, esqb_ref):
    i = pl.program_id(0)
    n_embed = et_ref.shape[1]
    dim = et_ref.shape[0]
    m = zt_ref.shape[1]

    @pl.when(i == 0)
    def _():
        e = jnp.transpose(et_ref[...])
        nrm = jnp.sqrt(jnp.sum(e * e, axis=1, keepdims=True))
        en = e / jnp.maximum(nrm, 1e-12)
        en_ref[:, 0:dim] = en
        esq = jnp.sum(en * en, axis=1, keepdims=True)
        esqb_ref[...] = jnp.broadcast_to(esq, (n_embed, m))

    z = jnp.transpose(zt_ref[...])
    zn_norm = jnp.sqrt(jnp.sum(z * z, axis=1, keepdims=True))
    zn = z / jnp.maximum(zn_norm, 1e-12)
    znt_ref[...] = jnp.transpose(zn)

    zsq = jnp.sum(zn * zn, axis=1, keepdims=True)
    zsqb = jnp.broadcast_to(jnp.transpose(zsq), (8, m))
    zm2 = -2.0 * zn

    n_chunks = n_embed // _CHUNK
    gpc = _CHUNK // 8

    rv = jnp.full((8, m), jnp.inf, jnp.float32)
    ri = jnp.zeros((8, m), jnp.int32)
    for c in range(n_chunks):
        en_c = en_ref[pl.ds(c * _CHUNK, _CHUNK), 0:dim]
        mm2 = lax.dot_general(
            en_c, zm2, (((1,), (1,)), ((), ())),
            preferred_element_type=jnp.float32)
        esq_c = esqb_ref[pl.ds(c * _CHUNK, _CHUNK), :]
        for g in range(gpc):
            t = c * gpc + g
            d = (esq_c[8 * g:8 * (g + 1), :] + zsqb) + mm2[8 * g:8 * (g + 1), :]
            better = d < rv
            ri = jnp.where(better, jnp.full((8, m), t, jnp.int32), ri)
            rv = jnp.minimum(rv, d)

    gid = ri * 8 + lax.broadcasted_iota(jnp.int32, (8, m), 0)
    gmin = jnp.min(rv, axis=0, keepdims=True)
    best_idx = jnp.min(jnp.where(rv == gmin, gid, n_embed),
                       axis=0, keepdims=True)
    idx2_ref[pl.ds(i, 1), :] = best_idx


def _dist_argmin(zt, et, nb):
    bd, m = zt.shape
    dim, n_embed = et.shape
    return pl.pallas_call(
        _dist_argmin_body,
        grid=(nb,),
        in_specs=[
            pl.BlockSpec((dim, m), lambda i: (i, 0)),
            pl.BlockSpec((dim, n_embed), lambda i: (0, 0)),
        ],
        out_specs=[
            pl.BlockSpec((dim, m), lambda i: (i, 0)),
            pl.BlockSpec((n_embed, _GPAD), lambda i: (0, 0)),
            pl.BlockSpec((nb, m), lambda i: (0, 0)),
        ],
        out_shape=[
            jax.ShapeDtypeStruct((bd, m), jnp.float32),
            jax.ShapeDtypeStruct((n_embed, _GPAD), jnp.float32),
            jax.ShapeDtypeStruct((nb, m), jnp.int32),
        ],
        scratch_shapes=[
            pltpu.VMEM((n_embed, m), jnp.float32),
        ],
    )(zt, et)


def _sc_gather(en, idx_flat):
    b = idx_flat.shape[0]
    dim = en.shape[1]
    info = plsc.get_sparse_core_info()
    nc, ns = info.num_cores, info.num_subcores
    nw = nc * ns
    b_per_w = b // nw

    mesh = plsc.VectorSubcoreMesh(core_axis_name="c", subcore_axis_name="s")

    @functools.partial(
        pl.kernel,
        mesh=mesh,
        out_type=jax.ShapeDtypeStruct((b, dim), jnp.float32),
        scratch_types=[
            pltpu.VMEM((b_per_w,), jnp.int32),
            pltpu.VMEM((b_per_w, dim), jnp.float32),
            pltpu.SemaphoreType.DMA,
        ],
    )
    def k(en_hbm, idx_hbm, out_hbm, idx_v, rows_v, sem):
        wid = lax.axis_index("s") * nc + lax.axis_index("c")
        base = wid * b_per_w
        pltpu.sync_copy(idx_hbm.at[pl.ds(base, b_per_w)], idx_v)
        pltpu.async_copy(en_hbm.at[idx_v], rows_v, sem).wait()
        pltpu.sync_copy(rows_v, out_hbm.at[pl.ds(base, b_per_w)])

    return k(en, idx_flat)


def _out_loss_body(zt_ref, znt_ref, zq_ref, out_ref, loss_ref):
    i = pl.program_id(0)
    nb = pl.num_programs(0)
    dim = zt_ref.shape[0]
    m = zt_ref.shape[1]
    zqt = jnp.transpose(zq_ref[:, 0:dim])
    zt = zt_ref[...]
    out_ref[...] = zt + (zqt - zt)
    diff = zqt - znt_ref[...]
    p = jnp.sum(diff * diff)
    tot = jnp.where(i == 0, p, loss_ref[0, 0] + p)
    mean = tot / jnp.float32(nb * dim * m)
    loss_ref[0, 0] = jnp.where(i == nb - 1, _BETA * mean + mean, tot)


def _out_loss(zt, znt, zq_pad, nb):
    bd, m = zt.shape
    dim = bd // nb
    return pl.pallas_call(
        _out_loss_body,
        grid=(nb,),
        in_specs=[
            pl.BlockSpec((dim, m), lambda i: (i, 0)),
            pl.BlockSpec((dim, m), lambda i: (i, 0)),
            pl.BlockSpec((m, _GPAD), lambda i: (i, 0)),
        ],
        out_specs=[
            pl.BlockSpec((dim, m), lambda i: (i, 0)),
            pl.BlockSpec(memory_space=pltpu.SMEM),
        ],
        out_shape=[
            jax.ShapeDtypeStruct((bd, m), jnp.float32),
            jax.ShapeDtypeStruct((1, 1), jnp.float32),
        ],
    )(zt, znt, zq_pad)


def kernel(z, embedding):
    b, t, dim = z.shape
    zt = jnp.transpose(z, (0, 2, 1)).reshape(b * dim, t)
    et = jnp.transpose(embedding)
    znt, en_pad, idx2 = _dist_argmin(zt, et, b)
    zq_pad = _sc_gather(en_pad, idx2.reshape(-1))
    out_t, loss = _out_loss(zt, znt, zq_pad, b)
    z_q_out = jnp.transpose(out_t.reshape(b, dim, t), (0, 2, 1))
    return (z_q_out, loss.reshape(()), idx2)


# speedup vs baseline: 1.9839x; 1.0454x over previous
import functools

import jax
import jax.numpy as jnp
from jax import lax
from jax.experimental import pallas as pl
from jax.experimental.pallas import tpu as pltpu
from jax.experimental.pallas import tpu_sc as plsc

_BETA = 0.25
_CHUNK = 2048
_GPAD = 128


def _dist_argmin_body(zt_ref, et_ref, znt_ref, en_ref, idx2_ref, esqb_ref):
    i = pl.program_id(0)
    n_embed = et_ref.shape[1]
    dim = et_ref.shape[0]
    m = zt_ref.shape[1]

    @pl.when(i == 0)
    def _():
        e = jnp.transpose(et_ref[...])
        nrm = jnp.sqrt(jnp.sum(e * e, axis=1, keepdims=True))
        en = e / jnp.maximum(nrm, 1e-12)
        en_ref[:, 0:dim] = en
        esq = jnp.sum(en * en, axis=1, keepdims=True)
        esqb_ref[...] = jnp.broadcast_to(esq, (n_embed, m))

    z = jnp.transpose(zt_ref[...])
    zn_norm = jnp.sqrt(jnp.sum(z * z, axis=1, keepdims=True))
    zn = z / jnp.maximum(zn_norm, 1e-12)
    znt_ref[...] = jnp.transpose(zn)

    zsq = jnp.sum(zn * zn, axis=1, keepdims=True)
    zsqb = jnp.broadcast_to(jnp.transpose(zsq), (8, m))
    zm2 = -2.0 * zn

    n_chunks = n_embed // _CHUNK
    gpc = _CHUNK // 8

    rv = [jnp.full((8, m), jnp.inf, jnp.float32) for _ in range(2)]
    ri = [jnp.zeros((8, m), jnp.int32) for _ in range(2)]
    for c in range(n_chunks):
        en_c = en_ref[pl.ds(c * _CHUNK, _CHUNK), 0:dim]
        mm2 = lax.dot_general(
            en_c, zm2, (((1,), (1,)), ((), ())),
            preferred_element_type=jnp.float32)
        esq_c = esqb_ref[pl.ds(c * _CHUNK, _CHUNK), :]
        for g in range(gpc):
            t = c * gpc + g
            k = t & 1
            d = (esq_c[8 * g:8 * (g + 1), :] + zsqb) + mm2[8 * g:8 * (g + 1), :]
            better = d < rv[k]
            ri[k] = jnp.where(better, jnp.full((8, m), t, jnp.int32), ri[k])
            rv[k] = jnp.minimum(rv[k], d)

    sub = lax.broadcasted_iota(jnp.int32, (8, m), 0)
    gid0 = ri[0] * 8 + sub
    gid1 = ri[1] * 8 + sub
    gmin = jnp.min(jnp.minimum(rv[0], rv[1]), axis=0, keepdims=True)
    cand0 = jnp.min(jnp.where(rv[0] == gmin, gid0, n_embed),
                    axis=0, keepdims=True)
    cand1 = jnp.min(jnp.where(rv[1] == gmin, gid1, n_embed),
                    axis=0, keepdims=True)
    best_idx = jnp.minimum(cand0, cand1)
    idx2_ref[pl.ds(i, 1), :] = best_idx


def _dist_argmin(zt, et, nb):
    bd, m = zt.shape
    dim, n_embed = et.shape
    return pl.pallas_call(
        _dist_argmin_body,
        grid=(nb,),
        in_specs=[
            pl.BlockSpec((dim, m), lambda i: (i, 0)),
            pl.BlockSpec((dim, n_embed), lambda i: (0, 0)),
        ],
        out_specs=[
            pl.BlockSpec((dim, m), lambda i: (i, 0)),
            pl.BlockSpec((n_embed, _GPAD), lambda i: (0, 0)),
            pl.BlockSpec((nb, m), lambda i: (0, 0)),
        ],
        out_shape=[
            jax.ShapeDtypeStruct((bd, m), jnp.float32),
            jax.ShapeDtypeStruct((n_embed, _GPAD), jnp.float32),
            jax.ShapeDtypeStruct((nb, m), jnp.int32),
        ],
        scratch_shapes=[
            pltpu.VMEM((n_embed, m), jnp.float32),
        ],
    )(zt, et)


def _sc_gather(en, idx_flat):
    b = idx_flat.shape[0]
    dim = en.shape[1]
    info = plsc.get_sparse_core_info()
    nc, ns = info.num_cores, info.num_subcores
    nw = nc * ns
    b_per_w = b // nw

    mesh = plsc.VectorSubcoreMesh(core_axis_name="c", subcore_axis_name="s")

    @functools.partial(
        pl.kernel,
        mesh=mesh,
        out_type=jax.ShapeDtypeStruct((b, dim), jnp.float32),
        scratch_types=[
            pltpu.VMEM((b_per_w,), jnp.int32),
            pltpu.VMEM((b_per_w, dim), jnp.float32),
            pltpu.SemaphoreType.DMA,
        ],
    )
    def k(en_hbm, idx_hbm, out_hbm, idx_v, rows_v, sem):
        wid = lax.axis_index("s") * nc + lax.axis_index("c")
        base = wid * b_per_w
        pltpu.sync_copy(idx_hbm.at[pl.ds(base, b_per_w)], idx_v)
        pltpu.async_copy(en_hbm.at[idx_v], rows_v, sem).wait()
        pltpu.sync_copy(rows_v, out_hbm.at[pl.ds(base, b_per_w)])

    return k(en, idx_flat)


def _out_loss_body(nb, zt_ref, znt_ref, zq_ref, out_ref, loss_ref):
    dim = zt_ref.shape[0] // nb
    m = zt_ref.shape[1]
    tot = jnp.float32(0.0)
    for b in range(nb):
        zqt = jnp.transpose(zq_ref[b * m:(b + 1) * m, 0:dim])
        zt = zt_ref[b * dim:(b + 1) * dim, :]
        out_ref[b * dim:(b + 1) * dim, :] = zt + (zqt - zt)
        diff = zqt - znt_ref[b * dim:(b + 1) * dim, :]
        tot = tot + jnp.sum(diff * diff)
    mean = tot / jnp.float32(nb * dim * m)
    loss_ref[0, 0] = _BETA * mean + mean


def _out_loss(zt, znt, zq_pad, nb):
    bd, m = zt.shape
    return pl.pallas_call(
        functools.partial(_out_loss_body, nb),
        out_specs=[
            pl.BlockSpec((bd, m), lambda: (0, 0)),
            pl.BlockSpec(memory_space=pltpu.SMEM),
        ],
        out_shape=[
            jax.ShapeDtypeStruct((bd, m), jnp.float32),
            jax.ShapeDtypeStruct((1, 1), jnp.float32),
        ],
    )(zt, znt, zq_pad)


def kernel(z, embedding):
    b, t, dim = z.shape
    zt = jnp.transpose(z, (0, 2, 1)).reshape(b * dim, t)
    et = jnp.transpose(embedding)
    znt, en_pad, idx2 = _dist_argmin(zt, et, b)
    zq_pad = _sc_gather(en_pad, idx2.reshape(-1))
    out_t, loss = _out_loss(zt, znt, zq_pad, b)
    z_q_out = jnp.transpose(out_t.reshape(b, dim, t), (0, 2, 1))
    return (z_q_out, loss.reshape(()), idx2)
